# baseline (speedup 1.0000x reference)
import functools

import jax
import jax.numpy as jnp
from jax import lax
from jax.experimental import pallas as pl
from jax.experimental.pallas import tpu as pltpu
from jax.experimental.pallas import tpu_sc as plsc

NC, NS, L = 2, 16, 16
NW = NC * NS
BATCH, HIST, D = 16384, 50, 128
B = BATCH * HIST
CB = 128
BPW = B // NW
NCHUNK = BPW // CB
NBUF = 4
ROUNDS = NCHUNK // NBUF


def _body(inp_hbm, table_hbm, out_hbm,
          xs_v, ys_v, zs_v, pos_v, idx_v, rows_v, *sems):
    csem = sems[0:NBUF]
    gsem = sems[NBUF:2 * NBUF]
    osem = sems[2 * NBUF:3 * NBUF]
    wid = lax.axis_index("s") * NC + lax.axis_index("c")
    base = wid * BPW
    lane3 = lax.iota(jnp.int32, L) * 3
    cbufs = (xs_v, ys_v, zs_v)

    for b in range(NBUF):
        for c in range(3):
            for j in range(CB // L):
                pos_v[b, c, pl.ds(j * L, L)] = (
                    lane3 + ((base + b * CB) * 3 + 48 * j + c))

    def fire_coords(b, bump):
        if bump:
            for c in range(3):
                for j in range(CB // L):
                    s = pl.ds(j * L, L)
                    pos_v[b, c, s] = pos_v[b, c, s] + (NBUF * CB * 3)
        for c in range(3):
            pltpu.async_copy(inp_hbm.at[pos_v.at[b, c]], cbufs[c].at[b],
                             csem[b])

    def wait_coords(b):
        for c in range(3):
            pltpu.make_async_copy(inp_hbm.at[pos_v.at[b, c]],
                                  cbufs[c].at[b], csem[b]).wait()

    def compute_idx(b):
        for j in range(CB // L):
            s = pl.ds(j * L, L)
            idx_v[b, s] = (xs_v[b, s] * 16 + ys_v[b, s]) * 16 + zs_v[b, s]

    def fire_gather(b):
        pltpu.async_copy(table_hbm.at[idx_v.at[b]], rows_v.at[b], gsem[b])

    def wait_gather(b):
        pltpu.make_async_copy(table_hbm.at[idx_v.at[b]], rows_v.at[b],
                              gsem[b]).wait()

    def fire_out(g, b):
        pltpu.async_copy(rows_v.at[b], out_hbm.at[pl.ds(base + g * CB, CB)],
                         osem[b])

    def wait_out(b):
        pltpu.make_async_copy(out_hbm.at[pl.ds(base, CB)], rows_v.at[b],
                              osem[b]).wait()

    for b in range(NBUF):
        fire_coords(b, bump=False)

    def round_body(r, carry):
        for b in range(NBUF):
            g = r * NBUF + b
            wait_coords(b)
            compute_idx(b)

            @pl.when(r > 0)
            def _():
                wait_out(b)

            fire_gather(b)
            pb = (b - 1) % NBUF
            if b > 0:
                wait_gather(pb)
                fire_out(g - 1, pb)
            else:
                @pl.when(r > 0)
                def _():
                    wait_gather(pb)
                    fire_out(g - 1, pb)

            @pl.when(r < ROUNDS - 1)
            def _():
                fire_coords(b, bump=True)
        return carry

    lax.fori_loop(0, ROUNDS, round_body, 0)

    bl = (NCHUNK - 1) % NBUF
    wait_gather(bl)
    pltpu.sync_copy(rows_v.at[bl], out_hbm.at[pl.ds(base + (NCHUNK - 1) * CB, CB)])
    for b in range(NBUF):
        if b != bl:
            wait_out(b)


_gather = functools.partial(
    pl.kernel,
    out_type=jax.ShapeDtypeStruct((B, D), jnp.float32),
    mesh=plsc.VectorSubcoreMesh(core_axis_name="c", subcore_axis_name="s"),
    scratch_types=(
        [
            pltpu.VMEM((NBUF, CB), jnp.int32),
            pltpu.VMEM((NBUF, CB), jnp.int32),
            pltpu.VMEM((NBUF, CB), jnp.int32),
            pltpu.VMEM((NBUF, 3, CB), jnp.int32),
            pltpu.VMEM((NBUF, CB), jnp.int32),
            pltpu.VMEM((NBUF, CB, D), jnp.float32),
        ]
        + [pltpu.SemaphoreType.DMA] * (3 * NBUF)
    ),
)(_body)


@jax.jit
def kernel(input, table):
    return _gather(input.reshape(B * 3), table).reshape(BATCH, HIST, D)


# speedup vs baseline: 1.0482x; 1.0482x over previous
import functools

import jax
import jax.numpy as jnp
from jax import lax
from jax.experimental import pallas as pl
from jax.experimental.pallas import tpu as pltpu
from jax.experimental.pallas import tpu_sc as plsc

NC, NS, L = 2, 16, 16
NW = NC * NS
BATCH, HIST, D = 16384, 50, 128
B = BATCH * HIST
CB = 128
BPW = B // NW
NCHUNK = BPW // CB
NBUF = 4
ROUNDS = NCHUNK // NBUF

IDX_ROWS = B // 128
IDX_BR = 640


def _idx_tc_body(inp_ref, out_ref):
    p = lax.broadcasted_iota(jnp.int32, (384, 128), 0)
    q = lax.broadcasted_iota(jnp.int32, (384, 128), 1)
    m = p - 3 * q
    w = jnp.where(m == 0, 256, jnp.where(m == 1, 16,
                                         jnp.where(m == 2, 1, 0)))
    x = inp_ref[...].astype(jnp.bfloat16)
    acc = lax.dot_general(x, w.astype(jnp.bfloat16), (((1,), (0,)), ((), ())),
                          preferred_element_type=jnp.float32)
    out_ref[...] = acc.astype(jnp.int32)


_idx_tc = pl.pallas_call(
    _idx_tc_body,
    grid=(IDX_ROWS // IDX_BR,),
    in_specs=[pl.BlockSpec((IDX_BR, 384), lambda i: (i, 0))],
    out_specs=pl.BlockSpec((IDX_BR, 128), lambda i: (i, 0)),
    out_shape=jax.ShapeDtypeStruct((IDX_ROWS, 128), jnp.int32),
)


def _body(idx_hbm, table_hbm, out_hbm, idx_v, rows_v, *sems):
    isem = sems[0:NBUF]
    gsem = sems[NBUF:2 * NBUF]
    osem = sems[2 * NBUF:3 * NBUF]
    wid = lax.axis_index("s") * NC + lax.axis_index("c")
    base = wid * BPW

    def fire_idx(g, b):
        pltpu.async_copy(idx_hbm.at[pl.ds(base + g * CB, CB)], idx_v.at[b],
                         isem[b])

    def wait_idx(b):
        pltpu.make_async_copy(idx_hbm.at[pl.ds(base, CB)], idx_v.at[b],
                              isem[b]).wait()

    def fire_gather(b):
        pltpu.async_copy(table_hbm.at[idx_v.at[b]], rows_v.at[b], gsem[b])

    def wait_gather(b):
        pltpu.make_async_copy(table_hbm.at[idx_v.at[b]], rows_v.at[b],
                              gsem[b]).wait()

    def fire_out(g, b):
        pltpu.async_copy(rows_v.at[b], out_hbm.at[pl.ds(base + g * CB, CB)],
                         osem[b])

    def wait_out(b):
        pltpu.make_async_copy(out_hbm.at[pl.ds(base, CB)], rows_v.at[b],
                              osem[b]).wait()

    for b in range(NBUF):
        fire_idx(b, b)

    def round_body(r, carry):
        for b in range(NBUF):
            g = r * NBUF + b
            wait_idx(b)

            @pl.when(r > 0)
            def _():
                wait_out(b)

            fire_gather(b)
            pb = (b - 1) % NBUF
            if b > 0:
                wait_gather(pb)
                fire_out(g - 1, pb)

                @pl.when(r < ROUNDS - 1)
                def _():
                    fire_idx(g - 1 + NBUF, pb)
            else:
                @pl.when(r > 0)
                def _():
                    wait_gather(pb)
                    fire_out(g - 1, pb)
                    fire_idx(g - 1 + NBUF, pb)
        return carry

    lax.fori_loop(0, ROUNDS, round_body, 0)

    bl = (NCHUNK - 1) % NBUF
    wait_gather(bl)
    pltpu.sync_copy(rows_v.at[bl], out_hbm.at[pl.ds(base + (NCHUNK - 1) * CB, CB)])
    for b in range(NBUF):
        if b != bl:
            wait_out(b)


_gather = functools.partial(
    pl.kernel,
    out_type=jax.ShapeDtypeStruct((B, D), jnp.float32),
    mesh=plsc.VectorSubcoreMesh(core_axis_name="c", subcore_axis_name="s"),
    scratch_types=(
        [
            pltpu.VMEM((NBUF, CB), jnp.int32),
            pltpu.VMEM((NBUF, CB, D), jnp.float32),
        ]
        + [pltpu.SemaphoreType.DMA] * (3 * NBUF)
    ),
)(_body)


@jax.jit
def kernel(input, table):
    idx = _idx_tc(input.reshape(IDX_ROWS, 384))
    return _gather(idx.reshape(B), table).reshape(BATCH, HIST, D)


# speedup vs baseline: 2.3073x; 2.2012x over previous
import functools

import jax
import jax.numpy as jnp
from jax import lax
from jax.experimental import pallas as pl
from jax.experimental.pallas import tpu as pltpu
from jax.experimental.pallas import tpu_sc as plsc

NC, NS, L = 2, 16, 16
NW = NC * NS
BATCH, HIST, D = 16384, 50, 128
B = BATCH * HIST
CB = 128
BPW = B // NW
NCHUNK = BPW // CB
NBUF = 4
ROUNDS = NCHUNK // NBUF

IDX_BR = 128


def _idx_tc_body(inp_ref, out_ref):
    x = inp_ref[:, :, 0]
    y = inp_ref[:, :, 1]
    z = inp_ref[:, :, 2]
    out_ref[...] = (x * 16 + y) * 16 + z


_idx_tc = pl.pallas_call(
    _idx_tc_body,
    grid=(BATCH // IDX_BR,),
    in_specs=[pl.BlockSpec((IDX_BR, HIST, 3), lambda i: (i, 0, 0))],
    out_specs=pl.BlockSpec((IDX_BR, HIST), lambda i: (i, 0)),
    out_shape=jax.ShapeDtypeStruct((BATCH, HIST), jnp.int32),
)


def _body(idx_hbm, table_hbm, out_hbm, idx_v, rows_v, *sems):
    isem = sems[0:NBUF]
    gsem = sems[NBUF:2 * NBUF]
    osem = sems[2 * NBUF:3 * NBUF]
    wid = lax.axis_index("s") * NC + lax.axis_index("c")
    base = wid * BPW

    def fire_idx(g, b):
        pltpu.async_copy(idx_hbm.at[pl.ds(base + g * CB, CB)], idx_v.at[b],
                         isem[b])

    def wait_idx(b):
        pltpu.make_async_copy(idx_hbm.at[pl.ds(base, CB)], idx_v.at[b],
                              isem[b]).wait()

    def fire_gather(b):
        pltpu.async_copy(table_hbm.at[idx_v.at[b]], rows_v.at[b], gsem[b])

    def wait_gather(b):
        pltpu.make_async_copy(table_hbm.at[idx_v.at[b]], rows_v.at[b],
                              gsem[b]).wait()

    def fire_out(g, b):
        pltpu.async_copy(rows_v.at[b], out_hbm.at[pl.ds(base + g * CB, CB)],
                         osem[b])

    def wait_out(b):
        pltpu.make_async_copy(out_hbm.at[pl.ds(base, CB)], rows_v.at[b],
                              osem[b]).wait()

    for b in range(NBUF):
        fire_idx(b, b)

    def round_body(r, carry):
        for b in range(NBUF):
            g = r * NBUF + b
            wait_idx(b)

            @pl.when(r > 0)
            def _():
                wait_out(b)

            fire_gather(b)
            pb = (b - 1) % NBUF
            if b > 0:
                wait_gather(pb)
                fire_out(g - 1, pb)

                @pl.when(r < ROUNDS - 1)
                def _():
                    fire_idx(g - 1 + NBUF, pb)
            else:
                @pl.when(r > 0)
                def _():
                    wait_gather(pb)
                    fire_out(g - 1, pb)
                    fire_idx(g - 1 + NBUF, pb)
        return carry

    lax.fori_loop(0, ROUNDS, round_body, 0)

    bl = (NCHUNK - 1) % NBUF
    wait_gather(bl)
    pltpu.sync_copy(rows_v.at[bl], out_hbm.at[pl.ds(base + (NCHUNK - 1) * CB, CB)])
    for b in range(NBUF):
        if b != bl:
            wait_out(b)


_gather = functools.partial(
    pl.kernel,
    out_type=jax.ShapeDtypeStruct((B, D), jnp.float32),
    mesh=plsc.VectorSubcoreMesh(core_axis_name="c", subcore_axis_name="s"),
    scratch_types=(
        [
            pltpu.VMEM((NBUF, CB), jnp.int32),
            pltpu.VMEM((NBUF, CB, D), jnp.float32),
        ]
        + [pltpu.SemaphoreType.DMA] * (3 * NBUF)
    ),
)(_body)


@jax.jit
def kernel(input, table):
    idx = _idx_tc(input)
    return _gather(idx.reshape(B), table).reshape(BATCH, HIST, D)


# speedup vs baseline: 5.7206x; 2.4794x over previous
import functools

import jax
import jax.numpy as jnp
from jax import lax
from jax.experimental import pallas as pl
from jax.experimental.pallas import tpu as pltpu
from jax.experimental.pallas import tpu_sc as plsc

NC, NS, L = 2, 16, 16
NW = NC * NS
BATCH, HIST, D = 16384, 50, 128
B = BATCH * HIST
BPWB = BATCH // NW
CHB = 4
LOOK = CHB * HIST
NCHUNK = BPWB // CHB
NBUF = 4
ROUNDS = NCHUNK // NBUF


def _body(xs_hbm, ys_hbm, zs_hbm, table_hbm, out_hbm,
          cx_v, cy_v, cz_v, idx_v, rows_v, *sems):
    csem = sems[0:NBUF]
    gsem = sems[NBUF:2 * NBUF]
    osem = sems[2 * NBUF:3 * NBUF]
    wid = lax.axis_index("s") * NC + lax.axis_index("c")
    batch0 = wid * BPWB
    i0 = wid * BPWB * HIST

    def fire_coords(g, b):
        s = pl.ds(i0 + g * LOOK, LOOK)
        d = pl.ds(b * LOOK, LOOK)
        pltpu.async_copy(xs_hbm.at[s], cx_v.at[d], csem[b])
        pltpu.async_copy(ys_hbm.at[s], cy_v.at[d], csem[b])
        pltpu.async_copy(zs_hbm.at[s], cz_v.at[d], csem[b])

    def wait_coords(b):
        d = pl.ds(b * LOOK, LOOK)
        for ref in (cx_v, cy_v, cz_v):
            pltpu.make_async_copy(xs_hbm.at[pl.ds(0, LOOK)], ref.at[d],
                                  csem[b]).wait()

    _OFFS = [16 * j for j in range(12)] + [LOOK - L]

    def compute_idx(b):
        for o in _OFFS:
            s = pl.ds(b * LOOK + o, L)
            idx_v[s] = (cx_v[s] * 16 + cy_v[s]) * 16 + cz_v[s]

    def fire_gather(b):
        pltpu.async_copy(table_hbm.at[idx_v.at[pl.ds(b * LOOK, 128)]],
                         rows_v.at[b, pl.ds(0, 128), :], gsem[b])
        pltpu.async_copy(table_hbm.at[idx_v.at[pl.ds(b * LOOK + 128, LOOK - 128)]],
                         rows_v.at[b, pl.ds(128, LOOK - 128), :], gsem[b])

    def wait_gather(b):
        pltpu.make_async_copy(table_hbm.at[idx_v.at[pl.ds(b * LOOK, 128)]],
                              rows_v.at[b, pl.ds(0, 128), :], gsem[b]).wait()
        pltpu.make_async_copy(table_hbm.at[idx_v.at[pl.ds(b * LOOK + 128, LOOK - 128)]],
                              rows_v.at[b, pl.ds(128, LOOK - 128), :],
                              gsem[b]).wait()

    def fire_out(g, b):
        for m in range(CHB):
            pltpu.async_copy(rows_v.at[b, pl.ds(m * HIST, HIST), :],
                             out_hbm.at[batch0 + g * CHB + m], osem[b])

    def wait_out(b):
        for m in range(CHB):
            pltpu.make_async_copy(rows_v.at[b, pl.ds(m * HIST, HIST), :],
                                  out_hbm.at[batch0], osem[b]).wait()

    for b in range(NBUF):
        fire_coords(b, b)

    def round_body(r, carry):
        for b in range(NBUF):
            g = r * NBUF + b
            wait_coords(b)
            compute_idx(b)

            @pl.when(r > 0)
            def _():
                wait_out(b)

            fire_gather(b)
            pb = (b - 1) % NBUF
            if b > 0:
                wait_gather(pb)
                fire_out(g - 1, pb)
            else:
                @pl.when(r > 0)
                def _():
                    wait_gather(pb)
                    fire_out(g - 1, pb)

            @pl.when(r < ROUNDS - 1)
            def _():
                fire_coords(g + NBUF, b)
        return carry

    lax.fori_loop(0, ROUNDS, round_body, 0)

    bl = (NCHUNK - 1) % NBUF
    wait_gather(bl)
    for m in range(CHB):
        pltpu.sync_copy(rows_v.at[bl, pl.ds(m * HIST, HIST), :],
                        out_hbm.at[batch0 + (NCHUNK - 1) * CHB + m])
    for b in range(NBUF):
        if b != bl:
            wait_out(b)


_gather = functools.partial(
    pl.kernel,
    out_type=jax.ShapeDtypeStruct((BATCH, HIST, D), jnp.float32),
    mesh=plsc.VectorSubcoreMesh(core_axis_name="c", subcore_axis_name="s"),
    scratch_types=(
        [
            pltpu.VMEM((NBUF * LOOK,), jnp.int32),
            pltpu.VMEM((NBUF * LOOK,), jnp.int32),
            pltpu.VMEM((NBUF * LOOK,), jnp.int32),
            pltpu.VMEM((NBUF * LOOK,), jnp.int32),
            pltpu.VMEM((NBUF, LOOK, D), jnp.float32),
        ]
        + [pltpu.SemaphoreType.DMA] * (3 * NBUF)
    ),
)(_body)


@jax.jit
def kernel(input, table):
    flat = input.reshape(B, 3)
    xs = flat[:, 0].reshape(B)
    ys = flat[:, 1].reshape(B)
    zs = flat[:, 2].reshape(B)
    return _gather(xs, ys, zs, table)
